# baseline (speedup 1.0000x reference)
import functools

import jax
import jax.numpy as jnp
import numpy as np
from jax import lax
from jax.experimental import pallas as pl
from jax.experimental.pallas import tpu as pltpu
from jax.experimental.pallas import tpu_sc as plsc

A_CMD = 10
B = 16384
TBL_ROWS = 768
BASE_INSTR = 128
BASE_CMD = 192
BASE_VAL = 224
BASE_TBLA = 480
BASE_TR = 512
D = 256
DH = 128
CHUNK = 128


def _dot_t(a, b):
    return lax.dot_general(a, b, (((1,), (1,)), ((), ())),
                           preferred_element_type=jnp.float32)


def _onehot(idx, width):
    cols = lax.broadcasted_iota(jnp.int32, (idx.shape[0], width), 1)
    return (idx[:, None] == cols).astype(jnp.float32)


def _prep_body(traces2_ref, tables2_ref, instr_meta_ref, synth_waves_ref,
               note_table_ref, w_helix_ref, hel_ref, fx_cmd_ref, fx_val_ref,
               w_fx_ref, w_table_ref, instr_table_ref, w_sw_ref, w_instr_ref,
               tparams_ref, cproj_ref, t_ref):
    fx_cmd = fx_cmd_ref[...]
    fx_val = fx_val_ref[...]
    w_fx = w_fx_ref[...]
    w_table = w_table_ref[...]

    gi = lax.broadcasted_iota(jnp.int32, (32, 512), 0)
    gj = lax.broadcasted_iota(jnp.int32, (32, 512), 1)
    avg = (gi == (gj >> 4)).astype(jnp.float32) * (1.0 / 16.0)

    tc = traces2_ref[:, 0]
    tv = traces2_ref[:, 1]
    row0 = jnp.concatenate(
        [jnp.dot(_onehot(tc, 32), fx_cmd, preferred_element_type=jnp.float32),
         jnp.dot(_onehot(tv, 256), fx_val, preferred_element_type=jnp.float32)],
        axis=-1)
    row0 = _dot_t(row0, w_fx)
    trace_emb = _dot_t(jnp.dot(avg, row0, preferred_element_type=jnp.float32),
                       w_table)

    kc = tables2_ref[:, 0]
    kv = tables2_ref[:, 1]
    val1 = jnp.where(
        (kc == A_CMD)[:, None],
        jnp.dot(_onehot(kv & 31, 32), trace_emb, preferred_element_type=jnp.float32),
        jnp.dot(_onehot(kv, 256), fx_val, preferred_element_type=jnp.float32))
    row1 = jnp.concatenate(
        [jnp.dot(_onehot(kc, 32), fx_cmd, preferred_element_type=jnp.float32),
         val1], axis=-1)
    row1 = _dot_t(row1, w_fx)
    table_emb = _dot_t(jnp.dot(avg, row1, preferred_element_type=jnp.float32),
                       w_table)

    note_emb = note_table_ref[...] + _dot_t(hel_ref[...], w_helix_ref[...])

    m0 = instr_meta_ref[:, 0]
    m1 = instr_meta_ref[:, 1]
    sw = _dot_t(jnp.dot(_onehot(m1, 16), synth_waves_ref[...],
                        preferred_element_type=jnp.float32), w_sw_ref[...])
    instr_feat = _dot_t(
        jnp.concatenate(
            [instr_table_ref[...],
             jnp.dot(_onehot(m0, 32), table_emb, preferred_element_type=jnp.float32),
             sw], axis=-1),
        w_instr_ref[...])

    w_fx_c = w_fx[:, 0:64]
    w_fx_v = w_fx[:, 64:128]
    cmdproj = _dot_t(fx_cmd, w_fx_c)
    valproj = _dot_t(fx_val, w_fx_v)
    tblaproj = _dot_t(table_emb, w_fx_v)

    tvals = lax.broadcasted_iota(jnp.int32, (256, 1), 0).astype(jnp.float32) \
        * (1.0 / 255.0)
    w1 = tparams_ref[0, :][None, :]
    b1 = tparams_ref[1, :][None, :]
    w2 = tparams_ref[2, :][None, :]
    b2 = tparams_ref[3, :][None, :]
    pre = tvals * w2 + b2
    trv = (tvals * w1 + b1) * (1.0 / (1.0 + jnp.exp(-pre)))

    ninth = 1.0 / 9.0
    for c in range(4):
        pc_ = cproj_ref[c]
        t_ref[c] = jnp.concatenate([
            _dot_t(note_emb, pc_[:, 0:128]),
            _dot_t(instr_feat, pc_[:, 128:256]),
            _dot_t(cmdproj, pc_[:, 256:384]) * ninth,
            _dot_t(valproj, pc_[:, 256:384]) * ninth,
            _dot_t(tblaproj, pc_[:, 256:384]) * ninth,
            _dot_t(trv, pc_[:, 384:400]),
        ], axis=0)


def _build_tables(tables_fx, traces_fx, instr_meta, synth_waves, note_table,
                  W_helix, fx_cmd_table, fx_val_table, W_fx, W_table,
                  instr_table, W_sw, W_instr, t_w1, t_b1, t_w2, t_b2,
                  channel_projections):
    n = np.arange(128, dtype=np.float32)
    ang = 2.0 * np.pi * n / 12.0
    hel = np.zeros((128, 8), dtype=np.float32)
    hel[:, 0] = np.cos(ang)
    hel[:, 1] = np.sin(ang)
    hel[:, 2] = n / 128.0
    hel = jnp.asarray(hel)
    w_helix_p = jnp.pad(W_helix, ((0, 0), (0, 5)))
    tparams = jnp.stack([t_w1, t_b1, t_w2, t_b2])
    return pl.pallas_call(
        _prep_body,
        out_shape=jax.ShapeDtypeStruct((4, TBL_ROWS, D), jnp.float32),
    )(traces_fx.reshape(512, 2), tables_fx.reshape(512, 2), instr_meta,
      synth_waves, note_table, w_helix_p, hel, fx_cmd_table, fx_val_table,
      W_fx, W_table, instr_table, W_sw, W_instr, tparams,
      channel_projections)


def _sc_body(t_hbm, steps_hbm, out_hbm, tbl, stp, outc):
    cid = lax.axis_index("c")
    sid = lax.axis_index("s")
    wid = sid * 2 + cid
    ch = wid & 3
    hf = (wid >> 2) & 1
    grp = wid >> 3

    tsz = TBL_ROWS * DH
    pltpu.sync_copy(
        t_hbm.at[pl.ds(pl.multiple_of((ch * 2 + hf) * tsz, 128), tsz)], tbl)

    iota16 = lax.broadcasted_iota(jnp.int32, (16,), 0)

    def chunk_body(chunk, _):
        row0 = pl.multiple_of(grp * (B // 4) + chunk * CHUNK, CHUNK)
        pltpu.sync_copy(steps_hbm.at[ch, :, pl.ds(row0, CHUNK)], stp)

        def g16_body(g, _):
            sl = pl.ds(g * 16, 16)
            ivm = [stp[0, sl] * DH, (stp[1, sl] + BASE_INSTR) * DH]
            for j in range(9):
                kc = 2 + 2 * j
                pc_ = stp[kc, sl]
                pv_ = stp[kc + 1, sl]
                ivm.append((pc_ + BASE_CMD) * DH)
                ivm.append(jnp.where(pc_ == A_CMD,
                                     (pv_ & 31) + BASE_TBLA,
                                     pv_ + BASE_VAL) * DH)
            ivm.append((stp[20, sl] + BASE_TR) * DH)

            rows = g * 16 + iota16

            def d_body(d, _):
                for u in range(4):
                    fd = (iota16 + (d * 4 + u)) & (DH - 1)
                    accs = [None, None, None]
                    for k in range(21):
                        v = plsc.load_gather(tbl, [ivm[k] + fd])
                        i = k % 3
                        accs[i] = v if accs[i] is None else accs[i] + v
                    plsc.store_scatter(outc, [rows, fd],
                                       (accs[0] + accs[1]) + accs[2])
                return 0

            lax.fori_loop(0, DH // 4, d_body, 0)
            return 0

        lax.fori_loop(0, CHUNK // 16, g16_body, 0)

        col0 = pl.multiple_of(ch * D + hf * DH, DH)
        pltpu.sync_copy(
            outc, out_hbm.at[pl.ds(row0, CHUNK), pl.ds(col0, DH)])
        return 0

    lax.fori_loop(0, (B // 4) // CHUNK, chunk_body, 0)


def _sc_gather_sum(t_sc, steps_t):
    mesh = plsc.VectorSubcoreMesh(core_axis_name="c", subcore_axis_name="s")
    return pl.kernel(
        _sc_body,
        mesh=mesh,
        compiler_params=pltpu.CompilerParams(needs_layout_passes=False),
        out_type=jax.ShapeDtypeStruct((B, 4 * D), jnp.float32),
        scratch_types=[
            pltpu.VMEM((TBL_ROWS * DH,), jnp.float32),
            pltpu.VMEM((21, CHUNK), jnp.int32),
            pltpu.VMEM((CHUNK, DH), jnp.float32),
        ],
    )(t_sc, steps_t)


def kernel(steps, tables_fx, traces_fx, instr_meta, synth_waves, note_table,
           W_helix, fx_cmd_table, fx_val_table, W_fx, W_table, instr_table,
           W_sw, W_instr, t_w1, t_b1, t_w2, t_b2, channel_projections):
    t_all = _build_tables(tables_fx, traces_fx, instr_meta, synth_waves,
                          note_table, W_helix, fx_cmd_table, fx_val_table,
                          W_fx, W_table, instr_table, W_sw, W_instr,
                          t_w1, t_b1, t_w2, t_b2, channel_projections)
    t_sc = t_all.reshape(4, TBL_ROWS, 2, DH).transpose(0, 2, 1, 3) \
                .reshape(4 * 2 * TBL_ROWS * DH)
    steps_t = steps.transpose(1, 2, 0)
    return _sc_gather_sum(t_sc, steps_t).reshape(B, 4, D)


# speedup vs baseline: 1.3887x; 1.3887x over previous
import functools

import jax
import jax.numpy as jnp
import numpy as np
from jax import lax
from jax.experimental import pallas as pl
from jax.experimental.pallas import tpu as pltpu
from jax.experimental.pallas import tpu_sc as plsc

A_CMD = 10
B = 16384
TBL_ROWS = 768
BASE_INSTR = 128
BASE_CMD = 192
BASE_VAL = 224
BASE_TBLA = 480
BASE_TR = 512
D = 256
DH = 128
CHUNK = 128


def _dot_t(a, b):
    return lax.dot_general(a, b, (((1,), (1,)), ((), ())),
                           preferred_element_type=jnp.float32)


def _onehot(idx, width):
    cols = lax.broadcasted_iota(jnp.int32, (idx.shape[0], width), 1)
    return (idx[:, None] == cols).astype(jnp.float32)


def _prep_body(traces2_ref, tables2_ref, instr_meta_ref, synth_waves_ref,
               note_table_ref, w_helix_ref, hel_ref, fx_cmd_ref, fx_val_ref,
               w_fx_ref, w_table_ref, instr_table_ref, w_sw_ref, w_instr_ref,
               tparams_ref, cproj_ref, t_ref):
    fx_cmd = fx_cmd_ref[...]
    fx_val = fx_val_ref[...]
    w_fx = w_fx_ref[...]
    w_table = w_table_ref[...]

    gi = lax.broadcasted_iota(jnp.int32, (32, 512), 0)
    gj = lax.broadcasted_iota(jnp.int32, (32, 512), 1)
    avg = (gi == (gj >> 4)).astype(jnp.float32) * (1.0 / 16.0)

    tc = traces2_ref[:, 0]
    tv = traces2_ref[:, 1]
    row0 = jnp.concatenate(
        [jnp.dot(_onehot(tc, 32), fx_cmd, preferred_element_type=jnp.float32),
         jnp.dot(_onehot(tv, 256), fx_val, preferred_element_type=jnp.float32)],
        axis=-1)
    row0 = _dot_t(row0, w_fx)
    trace_emb = _dot_t(jnp.dot(avg, row0, preferred_element_type=jnp.float32),
                       w_table)

    kc = tables2_ref[:, 0]
    kv = tables2_ref[:, 1]
    val1 = jnp.where(
        (kc == A_CMD)[:, None],
        jnp.dot(_onehot(kv & 31, 32), trace_emb, preferred_element_type=jnp.float32),
        jnp.dot(_onehot(kv, 256), fx_val, preferred_element_type=jnp.float32))
    row1 = jnp.concatenate(
        [jnp.dot(_onehot(kc, 32), fx_cmd, preferred_element_type=jnp.float32),
         val1], axis=-1)
    row1 = _dot_t(row1, w_fx)
    table_emb = _dot_t(jnp.dot(avg, row1, preferred_element_type=jnp.float32),
                       w_table)

    note_emb = note_table_ref[...] + _dot_t(hel_ref[...], w_helix_ref[...])

    m0 = instr_meta_ref[:, 0]
    m1 = instr_meta_ref[:, 1]
    sw = _dot_t(jnp.dot(_onehot(m1, 16), synth_waves_ref[...],
                        preferred_element_type=jnp.float32), w_sw_ref[...])
    instr_feat = _dot_t(
        jnp.concatenate(
            [instr_table_ref[...],
             jnp.dot(_onehot(m0, 32), table_emb, preferred_element_type=jnp.float32),
             sw], axis=-1),
        w_instr_ref[...])

    w_fx_c = w_fx[:, 0:64]
    w_fx_v = w_fx[:, 64:128]
    cmdproj = _dot_t(fx_cmd, w_fx_c)
    valproj = _dot_t(fx_val, w_fx_v)
    tblaproj = _dot_t(table_emb, w_fx_v)

    tvals = lax.broadcasted_iota(jnp.int32, (256, 1), 0).astype(jnp.float32) \
        * (1.0 / 255.0)
    w1 = tparams_ref[0, :][None, :]
    b1 = tparams_ref[1, :][None, :]
    w2 = tparams_ref[2, :][None, :]
    b2 = tparams_ref[3, :][None, :]
    pre = tvals * w2 + b2
    trv = (tvals * w1 + b1) * (1.0 / (1.0 + jnp.exp(-pre)))

    ninth = 1.0 / 9.0
    for c in range(4):
        pc_ = cproj_ref[c]
        t_ref[c] = jnp.concatenate([
            _dot_t(note_emb, pc_[:, 0:128]),
            _dot_t(instr_feat, pc_[:, 128:256]),
            _dot_t(cmdproj, pc_[:, 256:384]) * ninth,
            _dot_t(valproj, pc_[:, 256:384]) * ninth,
            _dot_t(tblaproj, pc_[:, 256:384]) * ninth,
            _dot_t(trv, pc_[:, 384:400]),
        ], axis=0)


def _build_tables(tables_fx, traces_fx, instr_meta, synth_waves, note_table,
                  W_helix, fx_cmd_table, fx_val_table, W_fx, W_table,
                  instr_table, W_sw, W_instr, t_w1, t_b1, t_w2, t_b2,
                  channel_projections):
    n = np.arange(128, dtype=np.float32)
    ang = 2.0 * np.pi * n / 12.0
    hel = np.zeros((128, 8), dtype=np.float32)
    hel[:, 0] = np.cos(ang)
    hel[:, 1] = np.sin(ang)
    hel[:, 2] = n / 128.0
    hel = jnp.asarray(hel)
    w_helix_p = jnp.pad(W_helix, ((0, 0), (0, 5)))
    tparams = jnp.stack([t_w1, t_b1, t_w2, t_b2])
    return pl.pallas_call(
        _prep_body,
        out_shape=jax.ShapeDtypeStruct((4, TBL_ROWS, D), jnp.float32),
    )(traces_fx.reshape(512, 2), tables_fx.reshape(512, 2), instr_meta,
      synth_waves, note_table, w_helix_p, hel, fx_cmd_table, fx_val_table,
      W_fx, W_table, instr_table, W_sw, W_instr, tparams,
      channel_projections)


DW = DH // 2


def _sc_body(t_hbm, steps_hbm, out_hbm, tbl, stp, outc):
    cid = lax.axis_index("c")
    sid = lax.axis_index("s")
    wid = sid * 2 + cid
    ch = wid & 3
    hf = (wid >> 2) & 1
    grp = wid >> 3

    tsz = TBL_ROWS * DW
    pltpu.sync_copy(
        t_hbm.at[pl.ds(pl.multiple_of((ch * 2 + hf) * tsz, 128), tsz)], tbl)

    iota16 = lax.broadcasted_iota(jnp.int32, (16,), 0)

    def chunk_body(chunk, _):
        row0 = pl.multiple_of(grp * (B // 4) + chunk * CHUNK, CHUNK)
        pltpu.sync_copy(steps_hbm.at[ch, :, pl.ds(row0, CHUNK)], stp)

        def g16_body(g, _):
            sl = pl.ds(g * 16, 16)
            ivm = [stp[0, sl] * DW, (stp[1, sl] + BASE_INSTR) * DW]
            for j in range(9):
                kc = 2 + 2 * j
                pc_ = stp[kc, sl]
                pv_ = stp[kc + 1, sl]
                ivm.append((pc_ + BASE_CMD) * DW)
                ivm.append(jnp.where(pc_ == A_CMD,
                                     (pv_ & 31) + BASE_TBLA,
                                     pv_ + BASE_VAL) * DW)
            ivm.append((stp[20, sl] + BASE_TR) * DW)

            rows = g * 16 + iota16

            def d_body(d, _):
                wd = (iota16 + d) & (DW - 1)
                ae = [None, None, None]
                ao = [None, None, None]
                for k in range(21):
                    v = plsc.load_gather(tbl, [ivm[k] + wd])
                    lo = plsc.bitcast(v << 16, jnp.float32)
                    hi = plsc.bitcast(v & jnp.int32(-65536), jnp.float32)
                    i = k % 3
                    ae[i] = lo if ae[i] is None else ae[i] + lo
                    ao[i] = hi if ao[i] is None else ao[i] + hi
                fe = wd * 2
                plsc.store_scatter(outc, [rows, fe], (ae[0] + ae[1]) + ae[2])
                plsc.store_scatter(outc, [rows, fe + 1],
                                   (ao[0] + ao[1]) + ao[2])
                return 0

            lax.fori_loop(0, DW, d_body, 0)
            return 0

        lax.fori_loop(0, CHUNK // 16, g16_body, 0)

        col0 = pl.multiple_of(ch * D + hf * DH, DH)
        pltpu.sync_copy(
            outc, out_hbm.at[pl.ds(row0, CHUNK), pl.ds(col0, DH)])
        return 0

    lax.fori_loop(0, (B // 4) // CHUNK, chunk_body, 0)


def _sc_gather_sum(t_sc, steps_t):
    mesh = plsc.VectorSubcoreMesh(core_axis_name="c", subcore_axis_name="s")
    return pl.kernel(
        _sc_body,
        mesh=mesh,
        compiler_params=pltpu.CompilerParams(needs_layout_passes=False),
        out_type=jax.ShapeDtypeStruct((B, 4 * D), jnp.float32),
        scratch_types=[
            pltpu.VMEM((TBL_ROWS * DW,), jnp.int32),
            pltpu.VMEM((21, CHUNK), jnp.int32),
            pltpu.VMEM((CHUNK, DH), jnp.float32),
        ],
    )(t_sc, steps_t)


def kernel(steps, tables_fx, traces_fx, instr_meta, synth_waves, note_table,
           W_helix, fx_cmd_table, fx_val_table, W_fx, W_table, instr_table,
           W_sw, W_instr, t_w1, t_b1, t_w2, t_b2, channel_projections):
    t_all = _build_tables(tables_fx, traces_fx, instr_meta, synth_waves,
                          note_table, W_helix, fx_cmd_table, fx_val_table,
                          W_fx, W_table, instr_table, W_sw, W_instr,
                          t_w1, t_b1, t_w2, t_b2, channel_projections)
    t_pk = jax.lax.bitcast_convert_type(
        t_all.astype(jnp.bfloat16).reshape(4, TBL_ROWS, 2, DW, 2), jnp.int32)
    t_sc = t_pk.transpose(0, 2, 1, 3).reshape(4 * 2 * TBL_ROWS * DW)
    steps_t = steps.transpose(1, 2, 0)
    return _sc_gather_sum(t_sc, steps_t).reshape(B, 4, D)


# speedup vs baseline: 4.2505x; 3.0607x over previous
import functools

import jax
import jax.numpy as jnp
import numpy as np
from jax import lax
from jax.experimental import pallas as pl
from jax.experimental.pallas import tpu as pltpu
from jax.experimental.pallas import tpu_sc as plsc

A_CMD = 10
B = 16384
TBL_ROWS = 768
BASE_INSTR = 128
BASE_CMD = 192
BASE_VAL = 224
BASE_TBLA = 480
BASE_TR = 512
D = 256
DH = 128
CHUNK = 128


def _dot_t(a, b):
    return lax.dot_general(a, b, (((1,), (1,)), ((), ())),
                           preferred_element_type=jnp.float32)


def _onehot(idx, width):
    cols = lax.broadcasted_iota(jnp.int32, (idx.shape[0], width), 1)
    return (idx[:, None] == cols).astype(jnp.float32)


def _prep_body(traces2_ref, tables2_ref, instr_meta_ref, synth_waves_ref,
               note_table_ref, w_helix_ref, hel_ref, fx_cmd_ref, fx_val_ref,
               w_fx_ref, w_table_ref, instr_table_ref, w_sw_ref, w_instr_ref,
               tparams_ref, cproj_ref, t_ref):
    fx_cmd = fx_cmd_ref[...]
    fx_val = fx_val_ref[...]
    w_fx = w_fx_ref[...]
    w_table = w_table_ref[...]

    gi = lax.broadcasted_iota(jnp.int32, (32, 512), 0)
    gj = lax.broadcasted_iota(jnp.int32, (32, 512), 1)
    avg = (gi == (gj >> 4)).astype(jnp.float32) * (1.0 / 16.0)

    tc = traces2_ref[:, 0]
    tv = traces2_ref[:, 1]
    row0 = jnp.concatenate(
        [jnp.dot(_onehot(tc, 32), fx_cmd, preferred_element_type=jnp.float32),
         jnp.dot(_onehot(tv, 256), fx_val, preferred_element_type=jnp.float32)],
        axis=-1)
    row0 = _dot_t(row0, w_fx)
    trace_emb = _dot_t(jnp.dot(avg, row0, preferred_element_type=jnp.float32),
                       w_table)

    kc = tables2_ref[:, 0]
    kv = tables2_ref[:, 1]
    val1 = jnp.where(
        (kc == A_CMD)[:, None],
        jnp.dot(_onehot(kv & 31, 32), trace_emb, preferred_element_type=jnp.float32),
        jnp.dot(_onehot(kv, 256), fx_val, preferred_element_type=jnp.float32))
    row1 = jnp.concatenate(
        [jnp.dot(_onehot(kc, 32), fx_cmd, preferred_element_type=jnp.float32),
         val1], axis=-1)
    row1 = _dot_t(row1, w_fx)
    table_emb = _dot_t(jnp.dot(avg, row1, preferred_element_type=jnp.float32),
                       w_table)

    note_emb = note_table_ref[...] + _dot_t(hel_ref[...], w_helix_ref[...])

    m0 = instr_meta_ref[:, 0]
    m1 = instr_meta_ref[:, 1]
    sw = _dot_t(jnp.dot(_onehot(m1, 16), synth_waves_ref[...],
                        preferred_element_type=jnp.float32), w_sw_ref[...])
    instr_feat = _dot_t(
        jnp.concatenate(
            [instr_table_ref[...],
             jnp.dot(_onehot(m0, 32), table_emb, preferred_element_type=jnp.float32),
             sw], axis=-1),
        w_instr_ref[...])

    w_fx_c = w_fx[:, 0:64]
    w_fx_v = w_fx[:, 64:128]
    cmdproj = _dot_t(fx_cmd, w_fx_c)
    valproj = _dot_t(fx_val, w_fx_v)
    tblaproj = _dot_t(table_emb, w_fx_v)

    tvals = lax.broadcasted_iota(jnp.int32, (256, 1), 0).astype(jnp.float32) \
        * (1.0 / 255.0)
    w1 = tparams_ref[0, :][None, :]
    b1 = tparams_ref[1, :][None, :]
    w2 = tparams_ref[2, :][None, :]
    b2 = tparams_ref[3, :][None, :]
    pre = tvals * w2 + b2
    trv = (tvals * w1 + b1) * (1.0 / (1.0 + jnp.exp(-pre)))

    ninth = 1.0 / 9.0
    for c in range(4):
        pc_ = cproj_ref[c]
        t_ref[c] = jnp.concatenate([
            _dot_t(note_emb, pc_[:, 0:128]),
            _dot_t(instr_feat, pc_[:, 128:256]),
            _dot_t(cmdproj, pc_[:, 256:384]) * ninth,
            _dot_t(valproj, pc_[:, 256:384]) * ninth,
            _dot_t(tblaproj, pc_[:, 256:384]) * ninth,
            _dot_t(trv, pc_[:, 384:400]),
        ], axis=0)


def _build_tables(tables_fx, traces_fx, instr_meta, synth_waves, note_table,
                  W_helix, fx_cmd_table, fx_val_table, W_fx, W_table,
                  instr_table, W_sw, W_instr, t_w1, t_b1, t_w2, t_b2,
                  channel_projections):
    n = np.arange(128, dtype=np.float32)
    ang = 2.0 * np.pi * n / 12.0
    hel = np.zeros((128, 8), dtype=np.float32)
    hel[:, 0] = np.cos(ang)
    hel[:, 1] = np.sin(ang)
    hel[:, 2] = n / 128.0
    hel = jnp.asarray(hel)
    w_helix_p = jnp.pad(W_helix, ((0, 0), (0, 5)))
    tparams = jnp.stack([t_w1, t_b1, t_w2, t_b2])
    return pl.pallas_call(
        _prep_body,
        out_shape=jax.ShapeDtypeStruct((4, TBL_ROWS, D), jnp.float32),
    )(traces_fx.reshape(512, 2), tables_fx.reshape(512, 2), instr_meta,
      synth_waves, note_table, w_helix_p, hel, fx_cmd_table, fx_val_table,
      W_fx, W_table, instr_table, W_sw, W_instr, tparams,
      channel_projections)


DW = DH // 2


def _sc_body(t_hbm, steps_hbm, out_hbm, tbl, stp, outc):
    cid = lax.axis_index("c")
    sid = lax.axis_index("s")
    wid = sid * 2 + cid
    ch = wid & 3
    hf = (wid >> 2) & 1
    grp = wid >> 3

    tsz = TBL_ROWS * DW
    pltpu.sync_copy(
        t_hbm.at[pl.ds(pl.multiple_of((ch * 2 + hf) * tsz, 128), tsz)], tbl)

    iota16 = lax.broadcasted_iota(jnp.int32, (16,), 0)

    def chunk_body(chunk, _):
        row0 = pl.multiple_of(grp * (B // 4) + chunk * CHUNK, CHUNK)
        pltpu.sync_copy(steps_hbm.at[ch, :, pl.ds(row0, CHUNK)], stp)

        def g16_body(g, _):
            sl = pl.ds(g * 16, 16)
            ivm = [stp[0, sl] * DW, (stp[1, sl] + BASE_INSTR) * DW]
            for j in range(9):
                kc = 2 + 2 * j
                pc_ = stp[kc, sl]
                pv_ = stp[kc + 1, sl]
                ivm.append((pc_ + BASE_CMD) * DW)
                ivm.append(jnp.where(pc_ == A_CMD,
                                     (pv_ & 31) + BASE_TBLA,
                                     pv_ + BASE_VAL) * DW)
            ivm.append((stp[20, sl] + BASE_TR) * DW)

            rows = g * 16 + iota16

            @functools.partial(plsc.parallel_loop, 0, DW, unroll=2)
            def d_body(d):
                wd = (iota16 + d) & (DW - 1)
                ae = [None, None, None]
                ao = [None, None, None]
                for k in range(21):
                    v = plsc.load_gather(tbl, [ivm[k] + wd])
                    lo = plsc.bitcast(v << 16, jnp.float32)
                    hi = plsc.bitcast(v & jnp.int32(-65536), jnp.float32)
                    i = k % 3
                    ae[i] = lo if ae[i] is None else ae[i] + lo
                    ao[i] = hi if ao[i] is None else ao[i] + hi
                fe = wd * 2
                plsc.store_scatter(outc, [rows, fe], (ae[0] + ae[1]) + ae[2])
                plsc.store_scatter(outc, [rows, fe + 1],
                                   (ao[0] + ao[1]) + ao[2])
            return 0

        lax.fori_loop(0, CHUNK // 16, g16_body, 0)

        col0 = pl.multiple_of(ch * D + hf * DH, DH)
        pltpu.sync_copy(
            outc, out_hbm.at[pl.ds(row0, CHUNK), pl.ds(col0, DH)])
        return 0

    lax.fori_loop(0, (B // 4) // CHUNK, chunk_body, 0)


def _sc_gather_sum(t_sc, steps_t):
    mesh = plsc.VectorSubcoreMesh(core_axis_name="c", subcore_axis_name="s")
    return pl.kernel(
        _sc_body,
        mesh=mesh,
        compiler_params=pltpu.CompilerParams(needs_layout_passes=False),
        out_type=jax.ShapeDtypeStruct((B, 4 * D), jnp.float32),
        scratch_types=[
            pltpu.VMEM((TBL_ROWS * DW,), jnp.int32),
            pltpu.VMEM((21, CHUNK), jnp.int32),
            pltpu.VMEM((CHUNK, DH), jnp.float32),
        ],
    )(t_sc, steps_t)


def kernel(steps, tables_fx, traces_fx, instr_meta, synth_waves, note_table,
           W_helix, fx_cmd_table, fx_val_table, W_fx, W_table, instr_table,
           W_sw, W_instr, t_w1, t_b1, t_w2, t_b2, channel_projections):
    t_all = _build_tables(tables_fx, traces_fx, instr_meta, synth_waves,
                          note_table, W_helix, fx_cmd_table, fx_val_table,
                          W_fx, W_table, instr_table, W_sw, W_instr,
                          t_w1, t_b1, t_w2, t_b2, channel_projections)
    t_pk = jax.lax.bitcast_convert_type(
        t_all.astype(jnp.bfloat16).reshape(4, TBL_ROWS, 2, DW, 2), jnp.int32)
    t_sc = t_pk.transpose(0, 2, 1, 3).reshape(4 * 2 * TBL_ROWS * DW)
    steps_t = steps.transpose(1, 2, 0)
    return _sc_gather_sum(t_sc, steps_t).reshape(B, 4, D)
